# Initial kernel scaffold; baseline (speedup 1.0000x reference)
import jax
import jax.numpy as jnp
from jax.experimental import pallas as pl


def kernel(x, edge_index, batch, W1, b1, W2, b2):
    raise NotImplementedError("write your pallas kernel here")


# speedup vs baseline: 7.9977x; 7.9977x over previous
import functools

import jax
import jax.numpy as jnp
from jax import lax
from jax.experimental import pallas as pl
from jax.experimental.pallas import tpu as pltpu
from jax.experimental.pallas import tpu_sc as plsc

G = 128
NPAD = 10240
CHUNK = 128
WIDE = 128
NC = 2
NS = 16
NW = NC * NS
RPT = NPAD // NS


def _edgesum_kernel(cpt: int, gather: bool):
    mesh = plsc.VectorSubcoreMesh(core_axis_name="c", subcore_axis_name="s")

    @functools.partial(
        pl.kernel,
        mesh=mesh,
        out_type=jax.ShapeDtypeStruct((NC, NPAD, WIDE), jnp.float32),
        scratch_types=[
            pltpu.VMEM_SHARED((NPAD, WIDE), jnp.float32),
            pltpu.VMEM((cpt, CHUNK), jnp.int32),
            pltpu.VMEM((cpt, CHUNK), jnp.int32),
            pltpu.VMEM((CHUNK, WIDE), jnp.float32),
        ],
    )
    def k(table_h, src_h, dst_h, zeros_h, out_h, acc, srcl, dstl, rows):
        c = lax.axis_index("c")
        s = lax.axis_index("s")
        wid = c * NS + s
        row0 = s * RPT

        @pl.when(c == 0)
        def _():
            pltpu.sync_copy(table_h.at[pl.ds(row0, RPT)], acc.at[pl.ds(row0, RPT)])

        @pl.when(c != 0)
        def _():
            pltpu.sync_copy(zeros_h, acc.at[pl.ds(row0, RPT)])

        if gather:
            pltpu.sync_copy(src_h.at[pl.ds(wid * cpt, cpt)], srcl)
        else:
            pltpu.sync_copy(table_h.at[pl.ds(0, CHUNK)], rows)
        pltpu.sync_copy(dst_h.at[pl.ds(wid * cpt, cpt)], dstl)
        plsc.subcore_barrier()

        def step(j, carry):
            if gather:
                pltpu.sync_copy(table_h.at[srcl.at[j]], rows)
            pltpu.sync_copy(rows, acc.at[dstl.at[j]], add=True)
            return carry

        lax.fori_loop(0, cpt, step, 0)
        plsc.subcore_barrier()
        pltpu.sync_copy(acc.at[pl.ds(row0, RPT)], out_h.at[c, pl.ds(row0, RPT)])

    return k


def _tc1_body(x_ref, w_ref, degp_ref, o_ref):
    deg = degp_ref[0, :, 0:1] + degp_ref[1, :, 0:1]
    dinv = lax.rsqrt(deg)
    h = jnp.dot(x_ref[...], w_ref[...], preferred_element_type=jnp.float32)
    o_ref[...] = h * dinv


def _tc2_body(agg_ref, degp_ref, b_ref, w_ref, o_ref):
    deg = degp_ref[0, :, 0:1] + degp_ref[1, :, 0:1]
    dinv = lax.rsqrt(deg)
    agg = agg_ref[0] + agg_ref[1]
    out1 = jnp.maximum(agg * dinv + b_ref[...], 0.0)
    h2 = jnp.dot(out1, w_ref[...], preferred_element_type=jnp.float32)
    blk, c_out = h2.shape
    o_ref[...] = jnp.pad(h2 * dinv, ((0, 0), (0, WIDE - c_out)))


def _tc3_body(agg_ref, degp_ref, b_ref, batch_ref, o_ref):
    c_out = b_ref.shape[1]
    deg = degp_ref[0, :, 0:1] + degp_ref[1, :, 0:1]
    dinv = lax.rsqrt(deg)
    agg = agg_ref[0, :, 0:c_out] + agg_ref[1, :, 0:c_out]
    out2 = agg * dinv + b_ref[...]
    gids = lax.broadcasted_iota(jnp.int32, (G, NPAD), 0)
    onehot = (batch_ref[...] == gids).astype(jnp.float32)
    sums = jnp.dot(onehot, out2, preferred_element_type=jnp.float32)
    counts = jnp.sum(onehot, axis=1, keepdims=True)
    o_ref[...] = sums / jnp.maximum(counts, 1.0)


def kernel(x, edge_index, batch, W1, b1, W2, b2):
    n, d = x.shape
    h = W1.shape[1]
    c_out = W2.shape[1]
    e = edge_index.shape[1]

    epad = -(-e // (CHUNK * NW * 8)) * (CHUNK * NW * 8)
    cpt = epad // (CHUNK * NW)
    nchunks = epad // CHUNK

    pad_e = epad - e
    pad_idx = jnp.full((pad_e,), n, dtype=jnp.int32)
    src_c = jnp.concatenate([edge_index[0], pad_idx]).reshape(nchunks, CHUNK)
    dst_c = jnp.concatenate([edge_index[1], pad_idx]).reshape(nchunks, CHUNK)

    x_pad = jnp.pad(x, ((0, NPAD - n), (0, 0)))
    batch_pad = jnp.pad(batch, (0, NPAD - n), constant_values=G).reshape(1, NPAD)
    ones_t = jnp.ones((NPAD, WIDE), dtype=jnp.float32)
    zeros_t = jnp.zeros((RPT, WIDE), dtype=jnp.float32)
    b1r = b1.reshape(1, h)
    b2r = b2.reshape(1, c_out)

    es_gather = _edgesum_kernel(cpt, gather=True)
    es_ones = _edgesum_kernel(cpt, gather=False)

    degp = es_ones(ones_t, src_c, dst_c, zeros_t)

    blk = 1024
    grid = NPAD // blk
    h1p = pl.pallas_call(
        _tc1_body,
        grid=(grid,),
        in_specs=[
            pl.BlockSpec((blk, d), lambda i: (i, 0)),
            pl.BlockSpec((d, h), lambda i: (0, 0)),
            pl.BlockSpec((NC, blk, WIDE), lambda i: (0, i, 0)),
        ],
        out_specs=pl.BlockSpec((blk, h), lambda i: (i, 0)),
        out_shape=jax.ShapeDtypeStruct((NPAD, h), jnp.float32),
    )(x_pad, W1, degp)

    agg1 = es_gather(h1p, src_c, dst_c, zeros_t)

    h2p = pl.pallas_call(
        _tc2_body,
        grid=(grid,),
        in_specs=[
            pl.BlockSpec((NC, blk, h), lambda i: (0, i, 0)),
            pl.BlockSpec((NC, blk, WIDE), lambda i: (0, i, 0)),
            pl.BlockSpec((1, h), lambda i: (0, 0)),
            pl.BlockSpec((h, c_out), lambda i: (0, 0)),
        ],
        out_specs=pl.BlockSpec((blk, WIDE), lambda i: (i, 0)),
        out_shape=jax.ShapeDtypeStruct((NPAD, WIDE), jnp.float32),
    )(agg1, degp, b1r, W2)

    agg2 = es_gather(h2p, src_c, dst_c, zeros_t)

    out = pl.pallas_call(
        _tc3_body,
        in_specs=[
            pl.BlockSpec((NC, NPAD, WIDE), lambda: (0, 0, 0)),
            pl.BlockSpec((NC, NPAD, WIDE), lambda: (0, 0, 0)),
            pl.BlockSpec((1, c_out), lambda: (0, 0)),
            pl.BlockSpec((1, NPAD), lambda: (0, 0)),
        ],
        out_specs=pl.BlockSpec((G, c_out), lambda: (0, 0)),
        out_shape=jax.ShapeDtypeStruct((G, c_out), jnp.float32),
    )(agg2, degp, b2r, batch_pad)

    return out


# speedup vs baseline: 19.8740x; 2.4850x over previous
import functools

import jax
import jax.numpy as jnp
from jax import lax
from jax.experimental import pallas as pl
from jax.experimental.pallas import tpu as pltpu
from jax.experimental.pallas import tpu_sc as plsc

G = 128
NPAD = 10240
CHUNK = 128
WIDE = 128
NC = 2
NS = 16
NW = NC * NS
RPT = NPAD // NS


def _edgesum_kernel(cpt: int, gather: bool):
    mesh = plsc.VectorSubcoreMesh(core_axis_name="c", subcore_axis_name="s")

    @functools.partial(
        pl.kernel,
        mesh=mesh,
        out_type=jax.ShapeDtypeStruct((NC, NPAD, WIDE), jnp.float32),
        scratch_types=[
            pltpu.VMEM_SHARED((NPAD, WIDE), jnp.float32),
            pltpu.VMEM((cpt, CHUNK), jnp.int32),
            pltpu.VMEM((cpt, CHUNK), jnp.int32),
            pltpu.VMEM((CHUNK, WIDE), jnp.float32),
        ],
    )
    def k(table_h, src_h, dst_h, zeros_h, out_h, acc, srcl, dstl, rows):
        c = lax.axis_index("c")
        s = lax.axis_index("s")
        wid = c * NS + s
        row0 = s * RPT

        @pl.when(c == 0)
        def _():
            pltpu.sync_copy(table_h.at[pl.ds(row0, RPT)], acc.at[pl.ds(row0, RPT)])

        @pl.when(c != 0)
        def _():
            pltpu.sync_copy(zeros_h, acc.at[pl.ds(row0, RPT)])

        if gather:
            pltpu.sync_copy(src_h.at[pl.ds(wid * cpt, cpt)], srcl)
        else:
            pltpu.sync_copy(table_h.at[pl.ds(0, CHUNK)], rows)
        pltpu.sync_copy(dst_h.at[pl.ds(wid * cpt, cpt)], dstl)
        plsc.subcore_barrier()

        def step(j, carry):
            if gather:
                pltpu.sync_copy(table_h.at[srcl.at[j]], rows)
            pltpu.sync_copy(rows, acc.at[dstl.at[j]], add=True)
            return carry

        lax.fori_loop(0, cpt, step, 0)
        plsc.subcore_barrier()
        pltpu.sync_copy(acc.at[pl.ds(row0, RPT)], out_h.at[c, pl.ds(row0, RPT)])

    return k


def _tc1_body(x_ref, w_ref, degp_ref, o_ref):
    deg = degp_ref[0, :, 0:1] + degp_ref[1, :, 0:1]
    dinv = lax.rsqrt(deg)
    h = jnp.dot(x_ref[...], w_ref[...], preferred_element_type=jnp.float32)
    o_ref[...] = h * dinv


def _tc2_body(agg_ref, degp_ref, b_ref, w_ref, o_ref):
    deg = degp_ref[0, :, 0:1] + degp_ref[1, :, 0:1]
    dinv = lax.rsqrt(deg)
    agg = agg_ref[0] + agg_ref[1]
    out1 = jnp.maximum(agg * dinv + b_ref[...], 0.0)
    h2 = jnp.dot(out1, w_ref[...], preferred_element_type=jnp.float32)
    blk, c_out = h2.shape
    o_ref[...] = jnp.pad(h2 * dinv, ((0, 0), (0, WIDE - c_out)))


def _tc3_body(agg_ref, degp_ref, b_ref, batch_ref, o_ref):
    c_out = b_ref.shape[1]
    deg = degp_ref[0, :, 0:1] + degp_ref[1, :, 0:1]
    dinv = lax.rsqrt(deg)
    agg = agg_ref[0, :, 0:c_out] + agg_ref[1, :, 0:c_out]
    out2 = agg * dinv + b_ref[...]
    gids = lax.broadcasted_iota(jnp.int32, (G, NPAD), 0)
    onehot = (batch_ref[...] == gids).astype(jnp.float32)
    sums = jnp.dot(onehot, out2, preferred_element_type=jnp.float32)
    counts = jnp.sum(onehot, axis=1, keepdims=True)
    o_ref[...] = sums / jnp.maximum(counts, 1.0)


def kernel(x, edge_index, batch, W1, b1, W2, b2):
    n, d = x.shape
    h = W1.shape[1]
    c_out = W2.shape[1]
    e = edge_index.shape[1]

    epad = -(-e // (CHUNK * NW * 8)) * (CHUNK * NW * 8)
    cpt = epad // (CHUNK * NW)
    nchunks = epad // CHUNK

    pad_e = epad - e
    pad_idx = n + jnp.arange(pad_e, dtype=jnp.int32) % (NPAD - n)
    src_c = jnp.concatenate([edge_index[0], pad_idx]).reshape(nchunks, CHUNK)
    dst_c = jnp.concatenate([edge_index[1], pad_idx]).reshape(nchunks, CHUNK)

    x_pad = jnp.pad(x, ((0, NPAD - n), (0, 0)))
    batch_pad = jnp.pad(batch, (0, NPAD - n), constant_values=G).reshape(1, NPAD)
    ones_t = jnp.ones((NPAD, WIDE), dtype=jnp.float32)
    zeros_t = jnp.zeros((RPT, WIDE), dtype=jnp.float32)
    b1r = b1.reshape(1, h)
    b2r = b2.reshape(1, c_out)

    es_gather = _edgesum_kernel(cpt, gather=True)
    es_ones = _edgesum_kernel(cpt, gather=False)

    degp = es_ones(ones_t, src_c, dst_c, zeros_t)

    blk = 1024
    grid = NPAD // blk
    h1p = pl.pallas_call(
        _tc1_body,
        grid=(grid,),
        in_specs=[
            pl.BlockSpec((blk, d), lambda i: (i, 0)),
            pl.BlockSpec((d, h), lambda i: (0, 0)),
            pl.BlockSpec((NC, blk, WIDE), lambda i: (0, i, 0)),
        ],
        out_specs=pl.BlockSpec((blk, h), lambda i: (i, 0)),
        out_shape=jax.ShapeDtypeStruct((NPAD, h), jnp.float32),
    )(x_pad, W1, degp)

    agg1 = es_gather(h1p, src_c, dst_c, zeros_t)

    h2p = pl.pallas_call(
        _tc2_body,
        grid=(grid,),
        in_specs=[
            pl.BlockSpec((NC, blk, h), lambda i: (0, i, 0)),
            pl.BlockSpec((NC, blk, WIDE), lambda i: (0, i, 0)),
            pl.BlockSpec((1, h), lambda i: (0, 0)),
            pl.BlockSpec((h, c_out), lambda i: (0, 0)),
        ],
        out_specs=pl.BlockSpec((blk, WIDE), lambda i: (i, 0)),
        out_shape=jax.ShapeDtypeStruct((NPAD, WIDE), jnp.float32),
    )(agg1, degp, b1r, W2)

    agg2 = es_gather(h2p, src_c, dst_c, zeros_t)

    out = pl.pallas_call(
        _tc3_body,
        in_specs=[
            pl.BlockSpec((NC, NPAD, WIDE), lambda: (0, 0, 0)),
            pl.BlockSpec((NC, NPAD, WIDE), lambda: (0, 0, 0)),
            pl.BlockSpec((1, c_out), lambda: (0, 0)),
            pl.BlockSpec((1, NPAD), lambda: (0, 0)),
        ],
        out_specs=pl.BlockSpec((G, c_out), lambda: (0, 0)),
        out_shape=jax.ShapeDtypeStruct((G, c_out), jnp.float32),
    )(agg2, degp, b2r, batch_pad)

    return out


# speedup vs baseline: 27.5943x; 1.3885x over previous
import functools

import jax
import jax.numpy as jnp
from jax import lax
from jax.experimental import pallas as pl
from jax.experimental.pallas import tpu as pltpu
from jax.experimental.pallas import tpu_sc as plsc

G = 128
NPAD = 10240
CHUNK = 128
WIDE = 128
NC = 2
NS = 16
NW = NC * NS
RPT = NPAD // NS


RING = 2
GRP = 8


def _edgesum_kernel(cpt: int, gather: bool):
    mesh = plsc.VectorSubcoreMesh(core_axis_name="c", subcore_axis_name="s")
    ngrp = cpt // GRP

    scratch = [
        pltpu.VMEM_SHARED((NPAD, WIDE), jnp.float32),
        pltpu.VMEM((cpt, CHUNK), jnp.int32),
        pltpu.VMEM((2, GRP, CHUNK), jnp.int32),
        pltpu.VMEM((RING if gather else 1, CHUNK, WIDE), jnp.float32),
        pltpu.SemaphoreType.DMA((RING,)),
        pltpu.SemaphoreType.DMA((RING,)),
        pltpu.SemaphoreType.DMA((2,)),
    ]

    @functools.partial(
        pl.kernel,
        mesh=mesh,
        out_type=jax.ShapeDtypeStruct((NC, NPAD, WIDE), jnp.float32),
        scratch_types=scratch,
    )
    def k(table_h, src_h, dst_h, zeros_h, out_h, acc, idxl, dstg, rows,
          gsem, ssem, dsem):
        c = lax.axis_index("c")
        s = lax.axis_index("s")
        wid = c * NS + s
        row0 = s * RPT
        base = wid * cpt

        @pl.when(c == 0)
        def _():
            pltpu.sync_copy(table_h.at[pl.ds(row0, RPT)], acc.at[pl.ds(row0, RPT)])

        @pl.when(c != 0)
        def _():
            pltpu.sync_copy(zeros_h, acc.at[pl.ds(row0, RPT)])

        if gather:
            pltpu.sync_copy(src_h.at[pl.ds(base, cpt)], idxl)
        else:
            pltpu.sync_copy(table_h.at[pl.ds(0, CHUNK)], rows.at[0])
            pltpu.sync_copy(dst_h.at[pl.ds(base, cpt)], idxl)
        plsc.subcore_barrier()

        if gather:
            def dst_load(g, q):
                return pltpu.async_copy(
                    dst_h.at[pl.ds(base + g * GRP, GRP)], dstg.at[q], dsem.at[q])

            dst_load(0, 0)
            dst_load(1, 1)
            for r in range(RING):
                pltpu.async_copy(table_h.at[idxl.at[r]], rows.at[r], gsem.at[r])

            def step(g, carry):
                q = g % 2
                pltpu.make_async_copy(
                    dst_h.at[pl.ds(base, GRP)], dstg.at[q], dsem.at[q]).wait()
                for t in range(GRP):
                    j = g * GRP + t
                    r = t % RING
                    pltpu.make_async_copy(
                        table_h.at[idxl.at[j]], rows.at[r], gsem.at[r]).wait()
                    pltpu.async_copy(rows.at[r], acc.at[dstg.at[q, t]],
                                     ssem.at[r], add=True)

                    @pl.when(j + RING < cpt)
                    def _():
                        pltpu.make_async_copy(
                            rows.at[r], acc.at[dstg.at[q, t]], ssem.at[r]).wait()
                        pltpu.async_copy(table_h.at[idxl.at[j + RING]],
                                         rows.at[r], gsem.at[r])

                @pl.when(g + 2 < ngrp)
                def _():
                    dst_load(g + 2, q)
                return carry

            lax.fori_loop(0, ngrp, step, 0)
            for r in range(RING):
                pltpu.make_async_copy(
                    rows.at[r], acc.at[dstg.at[0, 0]], ssem.at[r]).wait()
        else:

            def step(jj, carry):
                for r in range(RING):
                    j = jj * RING + r
                    pltpu.async_copy(rows.at[0], acc.at[idxl.at[j]],
                                     ssem.at[r], add=True)
                for r in range(RING):
                    j = jj * RING + r
                    pltpu.make_async_copy(
                        rows.at[0], acc.at[idxl.at[j]], ssem.at[r]).wait()
                return carry

            lax.fori_loop(0, cpt // RING, step, 0)

        plsc.subcore_barrier()
        pltpu.sync_copy(acc.at[pl.ds(row0, RPT)], out_h.at[c, pl.ds(row0, RPT)])

    return k


def _tc1_body(x_ref, w_ref, degp_ref, o_ref):
    deg = degp_ref[0, :, 0:1] + degp_ref[1, :, 0:1]
    dinv = lax.rsqrt(deg)
    h = jnp.dot(x_ref[...], w_ref[...], preferred_element_type=jnp.float32)
    o_ref[...] = h * dinv


def _tc2_body(agg_ref, degp_ref, b_ref, w_ref, o_ref):
    deg = degp_ref[0, :, 0:1] + degp_ref[1, :, 0:1]
    dinv = lax.rsqrt(deg)
    agg = agg_ref[0] + agg_ref[1]
    out1 = jnp.maximum(agg * dinv + b_ref[...], 0.0)
    h2 = jnp.dot(out1, w_ref[...], preferred_element_type=jnp.float32)
    blk, c_out = h2.shape
    o_ref[...] = jnp.pad(h2 * dinv, ((0, 0), (0, WIDE - c_out)))


def _tc3_body(agg_ref, degp_ref, b_ref, batch_ref, o_ref):
    c_out = b_ref.shape[1]
    deg = degp_ref[0, :, 0:1] + degp_ref[1, :, 0:1]
    dinv = lax.rsqrt(deg)
    agg = agg_ref[0, :, 0:c_out] + agg_ref[1, :, 0:c_out]
    out2 = agg * dinv + b_ref[...]
    gids = lax.broadcasted_iota(jnp.int32, (G, NPAD), 0)
    onehot = (batch_ref[...] == gids).astype(jnp.float32)
    sums = jnp.dot(onehot, out2, preferred_element_type=jnp.float32)
    counts = jnp.sum(onehot, axis=1, keepdims=True)
    o_ref[...] = sums / jnp.maximum(counts, 1.0)


def kernel(x, edge_index, batch, W1, b1, W2, b2):
    n, d = x.shape
    h = W1.shape[1]
    c_out = W2.shape[1]
    e = edge_index.shape[1]

    epad = -(-e // (CHUNK * NW * 8)) * (CHUNK * NW * 8)
    cpt = epad // (CHUNK * NW)
    nchunks = epad // CHUNK

    pad_e = epad - e
    pad_idx = n + jnp.arange(pad_e, dtype=jnp.int32) % (NPAD - n)
    src_c = jnp.concatenate([edge_index[0], pad_idx]).reshape(nchunks, CHUNK)
    dst_c = jnp.concatenate([edge_index[1], pad_idx]).reshape(nchunks, CHUNK)

    x_pad = jnp.pad(x, ((0, NPAD - n), (0, 0)))
    batch_pad = jnp.pad(batch, (0, NPAD - n), constant_values=G).reshape(1, NPAD)
    ones_t = jnp.ones((NPAD, WIDE), dtype=jnp.float32)
    zeros_t = jnp.zeros((RPT, WIDE), dtype=jnp.float32)
    b1r = b1.reshape(1, h)
    b2r = b2.reshape(1, c_out)

    es_gather = _edgesum_kernel(cpt, gather=True)
    es_ones = _edgesum_kernel(cpt, gather=False)

    degp = es_ones(ones_t, src_c, dst_c, zeros_t)

    blk = 1024
    grid = NPAD // blk
    h1p = pl.pallas_call(
        _tc1_body,
        grid=(grid,),
        in_specs=[
            pl.BlockSpec((blk, d), lambda i: (i, 0)),
            pl.BlockSpec((d, h), lambda i: (0, 0)),
            pl.BlockSpec((NC, blk, WIDE), lambda i: (0, i, 0)),
        ],
        out_specs=pl.BlockSpec((blk, h), lambda i: (i, 0)),
        out_shape=jax.ShapeDtypeStruct((NPAD, h), jnp.float32),
    )(x_pad, W1, degp)

    agg1 = es_gather(h1p, src_c, dst_c, zeros_t)

    h2p = pl.pallas_call(
        _tc2_body,
        grid=(grid,),
        in_specs=[
            pl.BlockSpec((NC, blk, h), lambda i: (0, i, 0)),
            pl.BlockSpec((NC, blk, WIDE), lambda i: (0, i, 0)),
            pl.BlockSpec((1, h), lambda i: (0, 0)),
            pl.BlockSpec((h, c_out), lambda i: (0, 0)),
        ],
        out_specs=pl.BlockSpec((blk, WIDE), lambda i: (i, 0)),
        out_shape=jax.ShapeDtypeStruct((NPAD, WIDE), jnp.float32),
    )(agg1, degp, b1r, W2)

    agg2 = es_gather(h2p, src_c, dst_c, zeros_t)

    out = pl.pallas_call(
        _tc3_body,
        in_specs=[
            pl.BlockSpec((NC, NPAD, WIDE), lambda: (0, 0, 0)),
            pl.BlockSpec((NC, NPAD, WIDE), lambda: (0, 0, 0)),
            pl.BlockSpec((1, c_out), lambda: (0, 0)),
            pl.BlockSpec((1, NPAD), lambda: (0, 0)),
        ],
        out_specs=pl.BlockSpec((G, c_out), lambda: (0, 0)),
        out_shape=jax.ShapeDtypeStruct((G, c_out), jnp.float32),
    )(agg2, degp, b2r, batch_pad)

    return out


# speedup vs baseline: 27.7225x; 1.0046x over previous
import functools

import jax
import jax.numpy as jnp
from jax import lax
from jax.experimental import pallas as pl
from jax.experimental.pallas import tpu as pltpu
from jax.experimental.pallas import tpu_sc as plsc

G = 128
NPAD = 10240
CHUNK = 128
WIDE = 128
NC = 2
NS = 16
NW = NC * NS
RPT = NPAD // NS


RING = 2
GRP = 8


def _edgesum_kernel(cpt: int, gather: bool):
    mesh = plsc.VectorSubcoreMesh(core_axis_name="c", subcore_axis_name="s")
    ngrp = cpt // GRP

    scratch = [
        pltpu.VMEM_SHARED((NPAD, WIDE), jnp.float32),
        pltpu.VMEM((cpt, CHUNK), jnp.int32),
        pltpu.VMEM((2, GRP, CHUNK), jnp.int32),
        pltpu.VMEM((RING if gather else 1, CHUNK, WIDE), jnp.float32),
        pltpu.SemaphoreType.DMA((RING,)),
        pltpu.SemaphoreType.DMA((RING,)),
        pltpu.SemaphoreType.DMA((2,)),
    ]

    @functools.partial(
        pl.kernel,
        mesh=mesh,
        out_type=jax.ShapeDtypeStruct((NC, NPAD, WIDE), jnp.float32),
        scratch_types=scratch,
    )
    def k(table_h, src_h, dst_h, zeros_h, out_h, acc, idxl, dstg, rows,
          gsem, ssem, dsem):
        c = lax.axis_index("c")
        s = lax.axis_index("s")
        wid = c * NS + s
        row0 = s * RPT
        base = wid * cpt

        @pl.when(c == 0)
        def _():
            t0 = row0 if gather else 0
            pltpu.sync_copy(table_h.at[pl.ds(t0, RPT)], acc.at[pl.ds(row0, RPT)])

        @pl.when(c != 0)
        def _():
            pltpu.sync_copy(zeros_h, acc.at[pl.ds(row0, RPT)])

        if gather:
            pltpu.sync_copy(src_h.at[pl.ds(base, cpt)], idxl)
        else:
            pltpu.sync_copy(table_h.at[pl.ds(0, CHUNK)], rows.at[0])
            pltpu.sync_copy(dst_h.at[pl.ds(base, cpt)], idxl)
        plsc.subcore_barrier()

        if gather:
            def dst_load(g, q):
                return pltpu.async_copy(
                    dst_h.at[pl.ds(base + g * GRP, GRP)], dstg.at[q], dsem.at[q])

            dst_load(0, 0)
            dst_load(1, 1)
            for r in range(RING):
                pltpu.async_copy(table_h.at[idxl.at[r]], rows.at[r], gsem.at[r])

            def step(g, carry):
                q = g % 2
                pltpu.make_async_copy(
                    dst_h.at[pl.ds(base, GRP)], dstg.at[q], dsem.at[q]).wait()
                for t in range(GRP):
                    j = g * GRP + t
                    r = t % RING
                    pltpu.make_async_copy(
                        table_h.at[idxl.at[j]], rows.at[r], gsem.at[r]).wait()
                    pltpu.async_copy(rows.at[r], acc.at[dstg.at[q, t]],
                                     ssem.at[r], add=True)

                    @pl.when(j + RING < cpt)
                    def _():
                        pltpu.make_async_copy(
                            rows.at[r], acc.at[dstg.at[q, t]], ssem.at[r]).wait()
                        pltpu.async_copy(table_h.at[idxl.at[j + RING]],
                                         rows.at[r], gsem.at[r])

                @pl.when(g + 2 < ngrp)
                def _():
                    dst_load(g + 2, q)
                return carry

            lax.fori_loop(0, ngrp, step, 0)
            for r in range(RING):
                pltpu.make_async_copy(
                    rows.at[r], acc.at[dstg.at[0, 0]], ssem.at[r]).wait()
        else:

            def step(jj, carry):
                for r in range(RING):
                    j = jj * RING + r
                    pltpu.async_copy(rows.at[0], acc.at[idxl.at[j]],
                                     ssem.at[r], add=True)
                for r in range(RING):
                    j = jj * RING + r
                    pltpu.make_async_copy(
                        rows.at[0], acc.at[idxl.at[j]], ssem.at[r]).wait()
                return carry

            lax.fori_loop(0, cpt // RING, step, 0)

        plsc.subcore_barrier()
        pltpu.sync_copy(acc.at[pl.ds(row0, RPT)], out_h.at[c, pl.ds(row0, RPT)])

    return k


def _tc1_body(x_ref, w_ref, degp_ref, o_ref):
    deg = degp_ref[0, :, 0:1] + degp_ref[1, :, 0:1]
    dinv = lax.rsqrt(deg)
    h = jnp.dot(x_ref[...], w_ref[...], preferred_element_type=jnp.float32)
    o_ref[...] = h * dinv


def _tc2_body(agg_ref, degp_ref, b_ref, w_ref, o_ref):
    deg = degp_ref[0, :, 0:1] + degp_ref[1, :, 0:1]
    dinv = lax.rsqrt(deg)
    agg = agg_ref[0] + agg_ref[1]
    out1 = jnp.maximum(agg * dinv + b_ref[...], 0.0)
    h2 = jnp.dot(out1, w_ref[...], preferred_element_type=jnp.float32)
    blk, c_out = h2.shape
    o_ref[...] = jnp.pad(h2 * dinv, ((0, 0), (0, WIDE - c_out)))


def _tc3_body(agg_ref, degp_ref, b_ref, batch_ref, o_ref):
    c_out = b_ref.shape[1]
    deg = degp_ref[0, :, 0:1] + degp_ref[1, :, 0:1]
    dinv = lax.rsqrt(deg)
    agg = agg_ref[0, :, 0:c_out] + agg_ref[1, :, 0:c_out]
    out2 = agg * dinv + b_ref[...]
    gids = lax.broadcasted_iota(jnp.int32, (G, NPAD), 0)
    onehot = (batch_ref[...] == gids).astype(jnp.float32)
    sums = jnp.dot(onehot, out2, preferred_element_type=jnp.float32)
    counts = jnp.sum(onehot, axis=1, keepdims=True)
    o_ref[...] = sums / jnp.maximum(counts, 1.0)


def kernel(x, edge_index, batch, W1, b1, W2, b2):
    n, d = x.shape
    h = W1.shape[1]
    c_out = W2.shape[1]
    e = edge_index.shape[1]

    epad = -(-e // (CHUNK * NW * 8)) * (CHUNK * NW * 8)
    cpt = epad // (CHUNK * NW)
    nchunks = epad // CHUNK

    pad_e = epad - e
    pad_idx = n + jnp.arange(pad_e, dtype=jnp.int32) % (NPAD - n)
    src_c = jnp.concatenate([edge_index[0], pad_idx]).reshape(nchunks, CHUNK)
    dst_c = jnp.concatenate([edge_index[1], pad_idx]).reshape(nchunks, CHUNK)

    x_pad = jnp.pad(x, ((0, NPAD - n), (0, 0)))
    batch_pad = jnp.pad(batch, (0, NPAD - n), constant_values=G).reshape(1, NPAD)
    ones_t = jnp.ones((RPT, WIDE), dtype=jnp.float32)
    zeros_t = jnp.zeros((RPT, WIDE), dtype=jnp.float32)
    b1r = b1.reshape(1, h)
    b2r = b2.reshape(1, c_out)

    es_gather = _edgesum_kernel(cpt, gather=True)
    es_ones = _edgesum_kernel(cpt, gather=False)

    degp = es_ones(ones_t, src_c, dst_c, zeros_t)

    blk = 1024
    grid = NPAD // blk
    h1p = pl.pallas_call(
        _tc1_body,
        grid=(grid,),
        in_specs=[
            pl.BlockSpec((blk, d), lambda i: (i, 0)),
            pl.BlockSpec((d, h), lambda i: (0, 0)),
            pl.BlockSpec((NC, blk, WIDE), lambda i: (0, i, 0)),
        ],
        out_specs=pl.BlockSpec((blk, h), lambda i: (i, 0)),
        out_shape=jax.ShapeDtypeStruct((NPAD, h), jnp.float32),
    )(x_pad, W1, degp)

    agg1 = es_gather(h1p, src_c, dst_c, zeros_t)

    h2p = pl.pallas_call(
        _tc2_body,
        grid=(grid,),
        in_specs=[
            pl.BlockSpec((NC, blk, h), lambda i: (0, i, 0)),
            pl.BlockSpec((NC, blk, WIDE), lambda i: (0, i, 0)),
            pl.BlockSpec((1, h), lambda i: (0, 0)),
            pl.BlockSpec((h, c_out), lambda i: (0, 0)),
        ],
        out_specs=pl.BlockSpec((blk, WIDE), lambda i: (i, 0)),
        out_shape=jax.ShapeDtypeStruct((NPAD, WIDE), jnp.float32),
    )(agg1, degp, b1r, W2)

    agg2 = es_gather(h2p, src_c, dst_c, zeros_t)

    out = pl.pallas_call(
        _tc3_body,
        in_specs=[
            pl.BlockSpec((NC, NPAD, WIDE), lambda: (0, 0, 0)),
            pl.BlockSpec((NC, NPAD, WIDE), lambda: (0, 0, 0)),
            pl.BlockSpec((1, c_out), lambda: (0, 0)),
            pl.BlockSpec((1, NPAD), lambda: (0, 0)),
        ],
        out_specs=pl.BlockSpec((G, c_out), lambda: (0, 0)),
        out_shape=jax.ShapeDtypeStruct((G, c_out), jnp.float32),
    )(agg2, degp, b2r, batch_pad)

    return out


# speedup vs baseline: 27.7597x; 1.0013x over previous
import functools

import jax
import jax.numpy as jnp
from jax import lax
from jax.experimental import pallas as pl
from jax.experimental.pallas import tpu as pltpu
from jax.experimental.pallas import tpu_sc as plsc

G = 128
NPAD = 10240
CHUNK = 128
WIDE = 128
NC = 2
NS = 16
NW = NC * NS
RPT = NPAD // NS


RING = 2
GRP = 8


def _edgesum_kernel(cpt: int, gather: bool):
    mesh = plsc.VectorSubcoreMesh(core_axis_name="c", subcore_axis_name="s")
    ngrp = cpt // GRP

    scratch = [
        pltpu.VMEM_SHARED((NPAD, WIDE), jnp.float32),
        pltpu.VMEM((cpt, CHUNK), jnp.int32),
        pltpu.VMEM((2, GRP, CHUNK), jnp.int32),
        pltpu.VMEM((RING if gather else 1, CHUNK, WIDE), jnp.float32),
        pltpu.SemaphoreType.DMA((RING,)),
        pltpu.SemaphoreType.DMA((RING,)),
        pltpu.SemaphoreType.DMA((2,)),
    ]

    @functools.partial(
        pl.kernel,
        mesh=mesh,
        out_type=jax.ShapeDtypeStruct((NC, NPAD, WIDE), jnp.float32),
        scratch_types=scratch,
    )
    def k(table_h, src_h, dst_h, zeros_h, out_h, acc, idxl, dstg, rows,
          gsem, ssem, dsem):
        c = lax.axis_index("c")
        s = lax.axis_index("s")
        wid = c * NS + s
        row0 = s * RPT
        base = wid * cpt

        @pl.when(c == 0)
        def _():
            t0 = row0 if gather else 0
            pltpu.sync_copy(table_h.at[pl.ds(t0, RPT)], acc.at[pl.ds(row0, RPT)])

        @pl.when(c != 0)
        def _():
            pltpu.sync_copy(zeros_h, acc.at[pl.ds(row0, RPT)])

        if gather:
            pltpu.sync_copy(src_h.at[pl.ds(base, cpt)], idxl)
        else:
            pltpu.sync_copy(table_h.at[pl.ds(0, CHUNK)], rows.at[0])
            pltpu.sync_copy(dst_h.at[pl.ds(base, cpt)], idxl)
        plsc.subcore_barrier()

        if gather:
            def dst_load(g, q):
                return pltpu.async_copy(
                    dst_h.at[pl.ds(base + g * GRP, GRP)], dstg.at[q], dsem.at[q])

            dst_load(0, 0)
            dst_load(1, 1)
            for r in range(RING):
                pltpu.async_copy(table_h.at[idxl.at[r]], rows.at[r], gsem.at[r])

            def step(g, carry):
                q = g % 2
                pltpu.make_async_copy(
                    dst_h.at[pl.ds(base, GRP)], dstg.at[q], dsem.at[q]).wait()
                for t in range(GRP):
                    j = g * GRP + t
                    r = t % RING
                    pltpu.make_async_copy(
                        table_h.at[idxl.at[j]], rows.at[r], gsem.at[r]).wait()
                    pltpu.async_copy(rows.at[r], acc.at[dstg.at[q, t]],
                                     ssem.at[r], add=True)

                    @pl.when(j + RING < cpt)
                    def _():
                        pltpu.make_async_copy(
                            rows.at[r], acc.at[dstg.at[q, t]], ssem.at[r]).wait()
                        pltpu.async_copy(table_h.at[idxl.at[j + RING]],
                                         rows.at[r], gsem.at[r])

                @pl.when(g + 2 < ngrp)
                def _():
                    dst_load(g + 2, q)
                return carry

            lax.fori_loop(0, ngrp, step, 0)
            for r in range(RING):
                pltpu.make_async_copy(
                    rows.at[r], acc.at[dstg.at[0, 0]], ssem.at[r]).wait()
        else:

            def step(jj, carry):
                for r in range(RING):
                    j = jj * RING + r
                    pltpu.async_copy(rows.at[0], acc.at[idxl.at[j]],
                                     ssem.at[r], add=True)
                for r in range(RING):
                    j = jj * RING + r
                    pltpu.make_async_copy(
                        rows.at[0], acc.at[idxl.at[j]], ssem.at[r]).wait()
                return carry

            lax.fori_loop(0, cpt // RING, step, 0)

        plsc.subcore_barrier()
        pltpu.sync_copy(acc.at[pl.ds(row0, RPT)], out_h.at[c, pl.ds(row0, RPT)])

    return k


def _tc1_body(x_ref, w_ref, degp_ref, o_ref, dinv_ref):
    deg = degp_ref[0, :, 0:1] + degp_ref[1, :, 0:1]
    dinv = lax.rsqrt(deg)
    h = jnp.dot(x_ref[...], w_ref[...], preferred_element_type=jnp.float32)
    o_ref[...] = h * dinv
    dinv_ref[...] = jnp.broadcast_to(dinv, dinv_ref.shape)


def _tc2_body(agg_ref, dinv16_ref, b_ref, w_ref, o_ref):
    dinv = dinv16_ref[:, 0:1]
    agg = agg_ref[0] + agg_ref[1]
    out1 = jnp.maximum(agg * dinv + b_ref[...], 0.0)
    h2 = jnp.dot(out1, w_ref[...], preferred_element_type=jnp.float32)
    blk, c_out = h2.shape
    o_ref[...] = jnp.pad(h2 * dinv, ((0, 0), (0, WIDE - c_out)))


def _tc3_body(agg_ref, dinv16_ref, b_ref, batch_ref, o_ref):
    c_out = b_ref.shape[1]
    dinv = dinv16_ref[:, 0:1]
    agg = agg_ref[0, :, 0:c_out] + agg_ref[1, :, 0:c_out]
    out2 = agg * dinv + b_ref[...]
    gids = lax.broadcasted_iota(jnp.int32, (G, NPAD), 0)
    onehot = (batch_ref[...] == gids).astype(jnp.float32)
    sums = jnp.dot(onehot, out2, preferred_element_type=jnp.float32)
    counts = jnp.sum(onehot, axis=1, keepdims=True)
    o_ref[...] = sums / jnp.maximum(counts, 1.0)


def kernel(x, edge_index, batch, W1, b1, W2, b2):
    n, d = x.shape
    h = W1.shape[1]
    c_out = W2.shape[1]
    e = edge_index.shape[1]

    epad = -(-e // (CHUNK * NW * 8)) * (CHUNK * NW * 8)
    cpt = epad // (CHUNK * NW)
    nchunks = epad // CHUNK

    pad_e = epad - e
    pad_idx = n + jnp.arange(pad_e, dtype=jnp.int32) % (NPAD - n)
    src_c = jnp.concatenate([edge_index[0], pad_idx]).reshape(nchunks, CHUNK)
    dst_c = jnp.concatenate([edge_index[1], pad_idx]).reshape(nchunks, CHUNK)

    x_pad = jnp.pad(x, ((0, NPAD - n), (0, 0)))
    batch_pad = jnp.pad(batch, (0, NPAD - n), constant_values=G).reshape(1, NPAD)
    ones_t = jnp.ones((RPT, WIDE), dtype=jnp.float32)
    zeros_t = jnp.zeros((RPT, WIDE), dtype=jnp.float32)
    b1r = b1.reshape(1, h)
    b2r = b2.reshape(1, c_out)

    es_gather = _edgesum_kernel(cpt, gather=True)
    es_ones = _edgesum_kernel(cpt, gather=False)

    degp = es_ones(ones_t, src_c, dst_c, zeros_t)

    blk = 1024
    grid = NPAD // blk
    h1p, dinv16 = pl.pallas_call(
        _tc1_body,
        grid=(grid,),
        in_specs=[
            pl.BlockSpec((blk, d), lambda i: (i, 0)),
            pl.BlockSpec((d, h), lambda i: (0, 0)),
            pl.BlockSpec((NC, blk, WIDE), lambda i: (0, i, 0)),
        ],
        out_specs=[
            pl.BlockSpec((blk, h), lambda i: (i, 0)),
            pl.BlockSpec((blk, 16), lambda i: (i, 0)),
        ],
        out_shape=[
            jax.ShapeDtypeStruct((NPAD, h), jnp.float32),
            jax.ShapeDtypeStruct((NPAD, 16), jnp.float32),
        ],
    )(x_pad, W1, degp)

    agg1 = es_gather(h1p, src_c, dst_c, zeros_t)

    h2p = pl.pallas_call(
        _tc2_body,
        grid=(grid,),
        in_specs=[
            pl.BlockSpec((NC, blk, h), lambda i: (0, i, 0)),
            pl.BlockSpec((blk, 16), lambda i: (i, 0)),
            pl.BlockSpec((1, h), lambda i: (0, 0)),
            pl.BlockSpec((h, c_out), lambda i: (0, 0)),
        ],
        out_specs=pl.BlockSpec((blk, WIDE), lambda i: (i, 0)),
        out_shape=jax.ShapeDtypeStruct((NPAD, WIDE), jnp.float32),
    )(agg1, dinv16, b1r, W2)

    agg2 = es_gather(h2p, src_c, dst_c, zeros_t)

    out = pl.pallas_call(
        _tc3_body,
        in_specs=[
            pl.BlockSpec((NC, NPAD, WIDE), lambda: (0, 0, 0)),
            pl.BlockSpec((NPAD, 16), lambda: (0, 0)),
            pl.BlockSpec((1, c_out), lambda: (0, 0)),
            pl.BlockSpec((1, NPAD), lambda: (0, 0)),
        ],
        out_specs=pl.BlockSpec((G, c_out), lambda: (0, 0)),
        out_shape=jax.ShapeDtypeStruct((G, c_out), jnp.float32),
    )(agg2, dinv16, b2r, batch_pad)

    return out


# speedup vs baseline: 28.2876x; 1.0190x over previous
import functools

import jax
import jax.numpy as jnp
from jax import lax
from jax.experimental import pallas as pl
from jax.experimental.pallas import tpu as pltpu
from jax.experimental.pallas import tpu_sc as plsc

G = 128
NPAD = 10240
CHUNK = 128
WIDE = 128
NC = 2
NS = 16
NW = NC * NS
RPT = NPAD // NS


RING = 2
GRP = 8


def _edgesum_kernel(cpt: int, gather: bool):
    mesh = plsc.VectorSubcoreMesh(core_axis_name="c", subcore_axis_name="s")
    ngrp = cpt // GRP

    scratch = [
        pltpu.VMEM_SHARED((NPAD, WIDE), jnp.float32),
        pltpu.VMEM((cpt, CHUNK), jnp.int32),
        pltpu.VMEM((2, GRP, CHUNK), jnp.int32),
        pltpu.VMEM((RING if gather else 1, CHUNK, WIDE), jnp.float32),
        pltpu.SemaphoreType.DMA((RING,)),
        pltpu.SemaphoreType.DMA((RING,)),
        pltpu.SemaphoreType.DMA((2,)),
    ]

    @functools.partial(
        pl.kernel,
        mesh=mesh,
        out_type=jax.ShapeDtypeStruct((NC, NPAD, WIDE), jnp.float32),
        scratch_types=scratch,
    )
    def k(table_h, ei_h, zeros_h, out_h, acc, idxl, dstg, rows,
          gsem, ssem, dsem):
        c = lax.axis_index("c")
        s = lax.axis_index("s")
        wid = c * NS + s
        row0 = s * RPT
        base = wid * cpt

        @pl.when(c == 0)
        def _():
            t0 = row0 if gather else 0
            pltpu.sync_copy(table_h.at[pl.ds(t0, RPT)], acc.at[pl.ds(row0, RPT)])

        @pl.when(c != 0)
        def _():
            pltpu.sync_copy(zeros_h, acc.at[pl.ds(row0, RPT)])

        if gather:
            pltpu.sync_copy(ei_h.at[0, pl.ds(base, cpt)], idxl)
        else:
            pltpu.sync_copy(table_h.at[pl.ds(0, CHUNK)], rows.at[0])
            pltpu.sync_copy(ei_h.at[1, pl.ds(base, cpt)], idxl)
        plsc.subcore_barrier()

        if gather:
            def dst_load(g, q):
                return pltpu.async_copy(
                    ei_h.at[1, pl.ds(base + g * GRP, GRP)], dstg.at[q], dsem.at[q])

            dst_load(0, 0)
            dst_load(1, 1)
            for r in range(RING):
                pltpu.async_copy(table_h.at[idxl.at[r]], rows.at[r], gsem.at[r])

            def step(g, carry):
                q = g % 2
                pltpu.make_async_copy(
                    ei_h.at[1, pl.ds(base, GRP)], dstg.at[q], dsem.at[q]).wait()
                for t in range(GRP):
                    j = g * GRP + t
                    r = t % RING
                    pltpu.make_async_copy(
                        table_h.at[idxl.at[j]], rows.at[r], gsem.at[r]).wait()
                    pltpu.async_copy(rows.at[r], acc.at[dstg.at[q, t]],
                                     ssem.at[r], add=True)

                    @pl.when(j + RING < cpt)
                    def _():
                        pltpu.make_async_copy(
                            rows.at[r], acc.at[dstg.at[q, t]], ssem.at[r]).wait()
                        pltpu.async_copy(table_h.at[idxl.at[j + RING]],
                                         rows.at[r], gsem.at[r])

                @pl.when(g + 2 < ngrp)
                def _():
                    dst_load(g + 2, q)
                return carry

            lax.fori_loop(0, ngrp, step, 0)
            for r in range(RING):
                pltpu.make_async_copy(
                    rows.at[r], acc.at[dstg.at[0, 0]], ssem.at[r]).wait()
        else:

            def step(jj, carry):
                for r in range(RING):
                    j = jj * RING + r
                    pltpu.async_copy(rows.at[0], acc.at[idxl.at[j]],
                                     ssem.at[r], add=True)
                for r in range(RING):
                    j = jj * RING + r
                    pltpu.make_async_copy(
                        rows.at[0], acc.at[idxl.at[j]], ssem.at[r]).wait()
                return carry

            lax.fori_loop(0, cpt // RING, step, 0)

        plsc.subcore_barrier()
        pltpu.sync_copy(acc.at[pl.ds(row0, RPT)], out_h.at[c, pl.ds(row0, RPT)])

    return k


def _tc1_body(x_ref, w_ref, degp_ref, o_ref, dinv_ref):
    deg = degp_ref[0, :, 0:1] + degp_ref[1, :, 0:1]
    dinv = lax.rsqrt(deg)
    h = jnp.dot(x_ref[...], w_ref[...], preferred_element_type=jnp.float32)
    o_ref[...] = h * dinv
    dinv_ref[...] = jnp.broadcast_to(dinv, dinv_ref.shape)


def _tc2_body(agg_ref, dinv16_ref, b_ref, w_ref, o_ref):
    dinv = dinv16_ref[:, 0:1]
    agg = agg_ref[0] + agg_ref[1]
    out1 = jnp.maximum(agg * dinv + b_ref[...], 0.0)
    h2 = jnp.dot(out1, w_ref[...], preferred_element_type=jnp.float32)
    blk, c_out = h2.shape
    o_ref[...] = jnp.pad(h2 * dinv, ((0, 0), (0, WIDE - c_out)))


def _tc3_body(agg_ref, dinv16_ref, b_ref, batch_ref, o_ref):
    c_out = b_ref.shape[1]
    dinv = dinv16_ref[:, 0:1]
    agg = agg_ref[0, :, 0:c_out] + agg_ref[1, :, 0:c_out]
    out2 = agg * dinv + b_ref[...]
    gids = lax.broadcasted_iota(jnp.int32, (G, NPAD), 0)
    onehot = (batch_ref[...] == gids).astype(jnp.float32)
    sums = jnp.dot(onehot, out2, preferred_element_type=jnp.float32)
    counts = jnp.sum(onehot, axis=1, keepdims=True)
    o_ref[...] = sums / jnp.maximum(counts, 1.0)


def kernel(x, edge_index, batch, W1, b1, W2, b2):
    n, d = x.shape
    h = W1.shape[1]
    c_out = W2.shape[1]
    e = edge_index.shape[1]

    epad = -(-e // (CHUNK * NW * 8)) * (CHUNK * NW * 8)
    cpt = epad // (CHUNK * NW)
    nchunks = epad // CHUNK

    pad_e = epad - e
    pad_idx = n + jnp.arange(pad_e, dtype=jnp.int32) % (NPAD - n)
    ei_c = jnp.concatenate(
        [edge_index, jnp.broadcast_to(pad_idx, (2, pad_e))], axis=1,
    ).reshape(2, nchunks, CHUNK)

    x_pad = jnp.pad(x, ((0, NPAD - n), (0, 0)))
    batch_pad = jnp.pad(batch, (0, NPAD - n), constant_values=G).reshape(1, NPAD)
    ones_t = jnp.ones((RPT, WIDE), dtype=jnp.float32)
    zeros_t = jnp.zeros((RPT, WIDE), dtype=jnp.float32)
    b1r = b1.reshape(1, h)
    b2r = b2.reshape(1, c_out)

    es_gather = _edgesum_kernel(cpt, gather=True)
    es_ones = _edgesum_kernel(cpt, gather=False)

    degp = es_ones(ones_t, ei_c, zeros_t)

    blk = 1024
    grid = NPAD // blk
    h1p, dinv16 = pl.pallas_call(
        _tc1_body,
        grid=(grid,),
        in_specs=[
            pl.BlockSpec((blk, d), lambda i: (i, 0)),
            pl.BlockSpec((d, h), lambda i: (0, 0)),
            pl.BlockSpec((NC, blk, WIDE), lambda i: (0, i, 0)),
        ],
        out_specs=[
            pl.BlockSpec((blk, h), lambda i: (i, 0)),
            pl.BlockSpec((blk, 16), lambda i: (i, 0)),
        ],
        out_shape=[
            jax.ShapeDtypeStruct((NPAD, h), jnp.float32),
            jax.ShapeDtypeStruct((NPAD, 16), jnp.float32),
        ],
    )(x_pad, W1, degp)

    agg1 = es_gather(h1p, ei_c, zeros_t)

    h2p = pl.pallas_call(
        _tc2_body,
        grid=(grid,),
        in_specs=[
            pl.BlockSpec((NC, blk, h), lambda i: (0, i, 0)),
            pl.BlockSpec((blk, 16), lambda i: (i, 0)),
            pl.BlockSpec((1, h), lambda i: (0, 0)),
            pl.BlockSpec((h, c_out), lambda i: (0, 0)),
        ],
        out_specs=pl.BlockSpec((blk, WIDE), lambda i: (i, 0)),
        out_shape=jax.ShapeDtypeStruct((NPAD, WIDE), jnp.float32),
    )(agg1, dinv16, b1r, W2)

    agg2 = es_gather(h2p, ei_c, zeros_t)

    out = pl.pallas_call(
        _tc3_body,
        in_specs=[
            pl.BlockSpec((NC, NPAD, WIDE), lambda: (0, 0, 0)),
            pl.BlockSpec((NPAD, 16), lambda: (0, 0)),
            pl.BlockSpec((1, c_out), lambda: (0, 0)),
            pl.BlockSpec((1, NPAD), lambda: (0, 0)),
        ],
        out_specs=pl.BlockSpec((G, c_out), lambda: (0, 0)),
        out_shape=jax.ShapeDtypeStruct((G, c_out), jnp.float32),
    )(agg2, dinv16, b2r, batch_pad)

    return out


# speedup vs baseline: 42.8496x; 1.5148x over previous
import functools

import jax
import jax.numpy as jnp
from jax import lax
from jax.experimental import pallas as pl
from jax.experimental.pallas import tpu as pltpu
from jax.experimental.pallas import tpu_sc as plsc

G = 128
NPAD = 10240
CHUNK = 128
WIDE = 128
NC = 2
NS = 16
NW = NC * NS
RPT = NPAD // NS


RING = 2
GRP = 8


def _edgesum_kernel(cpt: int, gather: bool, width: int = WIDE, ring: int = RING):
    mesh = plsc.VectorSubcoreMesh(core_axis_name="c", subcore_axis_name="s")
    ngrp = cpt // GRP
    RINGK = ring

    scratch = [
        pltpu.VMEM_SHARED((NPAD, width), jnp.float32),
        pltpu.VMEM((cpt, CHUNK), jnp.int32),
        pltpu.VMEM((2, GRP, CHUNK), jnp.int32),
        pltpu.VMEM((RINGK if gather else 1, CHUNK, width), jnp.float32),
        pltpu.SemaphoreType.DMA((RINGK,)),
        pltpu.SemaphoreType.DMA((RINGK,)),
        pltpu.SemaphoreType.DMA((2,)),
    ]

    @functools.partial(
        pl.kernel,
        mesh=mesh,
        out_type=jax.ShapeDtypeStruct((NC, NPAD, width), jnp.float32),
        compiler_params=pltpu.CompilerParams(use_tc_tiling_on_sc=False),
        scratch_types=scratch,
    )
    def k(table_h, ei_h, zeros_h, out_h, acc, idxl, dstg, rows,
          gsem, ssem, dsem):
        c = lax.axis_index("c")
        s = lax.axis_index("s")
        wid = c * NS + s
        row0 = s * RPT
        base = wid * cpt

        @pl.when(c == 0)
        def _():
            t0 = row0 if gather else 0
            pltpu.sync_copy(table_h.at[pl.ds(t0, RPT)], acc.at[pl.ds(row0, RPT)])

        @pl.when(c != 0)
        def _():
            pltpu.sync_copy(zeros_h, acc.at[pl.ds(row0, RPT)])

        if gather:
            pltpu.sync_copy(ei_h.at[0, pl.ds(base, cpt)], idxl)
        else:
            pltpu.sync_copy(table_h.at[pl.ds(0, CHUNK)], rows.at[0])
            pltpu.sync_copy(ei_h.at[1, pl.ds(base, cpt)], idxl)
        plsc.subcore_barrier()

        if gather:
            def dst_load(g, q):
                return pltpu.async_copy(
                    ei_h.at[1, pl.ds(base + g * GRP, GRP)], dstg.at[q], dsem.at[q])

            dst_load(0, 0)
            dst_load(1, 1)
            for r in range(RINGK):
                pltpu.async_copy(table_h.at[idxl.at[r]], rows.at[r], gsem.at[r])

            def step(g, carry):
                q = g % 2
                pltpu.make_async_copy(
                    ei_h.at[1, pl.ds(base, GRP)], dstg.at[q], dsem.at[q]).wait()
                for t in range(GRP):
                    j = g * GRP + t
                    r = t % RINGK
                    pltpu.make_async_copy(
                        table_h.at[idxl.at[j]], rows.at[r], gsem.at[r]).wait()
                    pltpu.async_copy(rows.at[r], acc.at[dstg.at[q, t]],
                                     ssem.at[r], add=True)

                    @pl.when(j + RINGK < cpt)
                    def _():
                        pltpu.make_async_copy(
                            rows.at[r], acc.at[dstg.at[q, t]], ssem.at[r]).wait()
                        pltpu.async_copy(table_h.at[idxl.at[j + RINGK]],
                                         rows.at[r], gsem.at[r])

                @pl.when(g + 2 < ngrp)
                def _():
                    dst_load(g + 2, q)
                return carry

            lax.fori_loop(0, ngrp, step, 0)
            for r in range(RINGK):
                pltpu.make_async_copy(
                    rows.at[r], acc.at[dstg.at[0, 0]], ssem.at[r]).wait()
        else:

            def step(jj, carry):
                for r in range(RINGK):
                    j = jj * RINGK + r
                    pltpu.async_copy(rows.at[0], acc.at[idxl.at[j]],
                                     ssem.at[r], add=True)
                for r in range(RINGK):
                    j = jj * RINGK + r
                    pltpu.make_async_copy(
                        rows.at[0], acc.at[idxl.at[j]], ssem.at[r]).wait()
                return carry

            lax.fori_loop(0, cpt // RINGK, step, 0)

        plsc.subcore_barrier()
        pltpu.sync_copy(acc.at[pl.ds(row0, RPT)], out_h.at[c, pl.ds(row0, RPT)])

    return k


def _tc1_body(x_ref, w_ref, degp_ref, o_ref, dinv_ref):
    deg = degp_ref[0, :, 0:1] + degp_ref[1, :, 0:1]

    dinv = lax.rsqrt(deg)
    h = jnp.dot(x_ref[...], w_ref[...], preferred_element_type=jnp.float32)
    o_ref[...] = h * dinv
    dinv_ref[...] = jnp.broadcast_to(dinv, dinv_ref.shape)


def _tc2_body(agg_ref, dinv16_ref, b_ref, w_ref, o_ref):
    dinv = dinv16_ref[:, 0:1]
    agg = agg_ref[0] + agg_ref[1]
    out1 = jnp.maximum(agg * dinv + b_ref[...], 0.0)
    h2 = jnp.dot(out1, w_ref[...], preferred_element_type=jnp.float32)
    o_ref[...] = h2 * dinv


def _tc3_body(agg_ref, dinv16_ref, b_ref, batch_ref, o_ref):
    dinv = dinv16_ref[:, 0:1]
    agg = agg_ref[0] + agg_ref[1]
    out2 = agg * dinv + b_ref[...]
    gids = lax.broadcasted_iota(jnp.int32, (G, NPAD), 0)
    onehot = (batch_ref[...] == gids).astype(jnp.float32)
    sums = jnp.dot(onehot, out2, preferred_element_type=jnp.float32)
    counts = jnp.sum(onehot, axis=1, keepdims=True)
    o_ref[...] = sums / jnp.maximum(counts, 1.0)


def kernel(x, edge_index, batch, W1, b1, W2, b2):
    n, d = x.shape
    h = W1.shape[1]
    c_out = W2.shape[1]
    e = edge_index.shape[1]

    epad = -(-e // (CHUNK * NW * 8)) * (CHUNK * NW * 8)
    cpt = epad // (CHUNK * NW)
    nchunks = epad // CHUNK

    pad_e = epad - e
    pad_idx = n + jnp.arange(pad_e, dtype=jnp.int32) % (NPAD - n)
    ei_c = jnp.concatenate(
        [edge_index, jnp.broadcast_to(pad_idx, (2, pad_e))], axis=1,
    ).reshape(2, nchunks, CHUNK)

    x_pad = jnp.pad(x, ((0, NPAD - n), (0, 0)))
    batch_pad = jnp.pad(batch, (0, NPAD - n), constant_values=G).reshape(1, NPAD)
    ones_c = jnp.ones((RPT, c_out), dtype=jnp.float32)
    zeros_c = jnp.zeros((RPT, c_out), dtype=jnp.float32)
    zeros_t = jnp.zeros((RPT, WIDE), dtype=jnp.float32)
    b1r = b1.reshape(1, h)
    b2r = b2.reshape(1, c_out)

    es_wide = _edgesum_kernel(cpt, gather=True, width=h, ring=RING)
    es_narrow = _edgesum_kernel(cpt, gather=True, width=c_out, ring=4)
    es_ones = _edgesum_kernel(cpt, gather=False, width=c_out, ring=4)

    degp = es_ones(ones_c, ei_c, zeros_c)

    blk = 1024
    grid = NPAD // blk
    h1p, dinv16 = pl.pallas_call(
        _tc1_body,
        grid=(grid,),
        in_specs=[
            pl.BlockSpec((blk, d), lambda i: (i, 0)),
            pl.BlockSpec((d, h), lambda i: (0, 0)),
            pl.BlockSpec((NC, blk, 16), lambda i: (0, i, 0)),
        ],
        out_specs=[
            pl.BlockSpec((blk, h), lambda i: (i, 0)),
            pl.BlockSpec((blk, 16), lambda i: (i, 0)),
        ],
        out_shape=[
            jax.ShapeDtypeStruct((NPAD, h), jnp.float32),
            jax.ShapeDtypeStruct((NPAD, 16), jnp.float32),
        ],
    )(x_pad, W1, degp)

    agg1 = es_wide(h1p, ei_c, zeros_t)

    h2p = pl.pallas_call(
        _tc2_body,
        grid=(grid,),
        in_specs=[
            pl.BlockSpec((NC, blk, h), lambda i: (0, i, 0)),
            pl.BlockSpec((blk, 16), lambda i: (i, 0)),
            pl.BlockSpec((1, h), lambda i: (0, 0)),
            pl.BlockSpec((h, c_out), lambda i: (0, 0)),
        ],
        out_specs=pl.BlockSpec((blk, c_out), lambda i: (i, 0)),
        out_shape=jax.ShapeDtypeStruct((NPAD, c_out), jnp.float32),
    )(agg1, dinv16, b1r, W2)

    agg2 = es_narrow(h2p, ei_c, zeros_c)

    out = pl.pallas_call(
        _tc3_body,
        in_specs=[
            pl.BlockSpec((NC, NPAD, c_out), lambda: (0, 0, 0)),
            pl.BlockSpec((NPAD, 16), lambda: (0, 0)),
            pl.BlockSpec((1, c_out), lambda: (0, 0)),
            pl.BlockSpec((1, NPAD), lambda: (0, 0)),
        ],
        out_specs=pl.BlockSpec((G, c_out), lambda: (0, 0)),
        out_shape=jax.ShapeDtypeStruct((G, c_out), jnp.float32),
    )(agg2, dinv16, b2r, batch_pad)

    return out
